# baseline (speedup 1.0000x reference)
import functools

import jax
import jax.numpy as jnp
import numpy as np
from jax import lax
from jax.experimental import pallas as pl
from jax.experimental.pallas import tpu as pltpu
from jax.experimental.pallas import tpu_sc as plsc

_ROWS_PER_BLOCK = 512
_LOW_BIT = 12
_SIGN_BIT = np.int32(-(2**31))
_REST_MASK = np.int32(0x7FFFFFFF)

_SC_ROWS = 4096
_SC_BATCH = 16


def _median_offset_block(x_ref, o_ref, *, n_cols, low_bit):
    xb = x_ref[...]
    bits = jax.lax.bitcast_convert_type(xb, jnp.int32)
    key = bits ^ ((bits >> 31) & _REST_MASK)
    k = np.float32((n_cols - 1) // 2)

    def count_below(mask):
        return jnp.sum(mask.astype(jnp.float32), axis=1, keepdims=True)

    c = count_below(key < 0)
    p = jnp.where(c > k, _SIGN_BIT, np.int32(0))
    for bit in range(30, low_bit - 1, -1):
        mid = p + np.int32(1 << bit)
        c = count_below(key < mid)
        p = jnp.where(c > k, p, mid)

    med_bits = jnp.where(p < 0, p ^ _REST_MASK, p)
    med = jax.lax.bitcast_convert_type(med_bits, jnp.float32)
    o_ref[...] = xb - med


def _tc_part(x, skip_rows):
    m, n = x.shape
    r = _ROWS_PER_BLOCK
    off = skip_rows // r
    body = functools.partial(_median_offset_block, n_cols=n, low_bit=_LOW_BIT)
    return pl.pallas_call(
        body,
        grid=(m // r - off,),
        in_specs=[pl.BlockSpec((r, n), lambda i: (i + off, 0))],
        out_specs=pl.BlockSpec((r, n), lambda i: (i + off, 0)),
        out_shape=jax.ShapeDtypeStruct((m, n), x.dtype),
        compiler_params=pltpu.CompilerParams(
            dimension_semantics=("arbitrary",)),
    )(x)


def _sc_part(x):
    _, n = x.shape
    s = _SC_ROWS
    info = plsc.get_sparse_core_info()
    nc, ns, l = info.num_cores, info.num_subcores, info.num_lanes
    nw = nc * ns
    rows_w = s // nw
    nb = rows_w // _SC_BATCH
    k = np.int32((n - 1) // 2)
    nchunk = n // l
    unroll = 8
    mesh = plsc.VectorSubcoreMesh(core_axis_name="c", subcore_axis_name="s")

    @functools.partial(
        pl.kernel, mesh=mesh,
        out_type=jax.ShapeDtypeStruct((s, n), jnp.float32),
        compiler_params=pltpu.CompilerParams(
            needs_layout_passes=False, use_tc_tiling_on_sc=False),
        scratch_types=[
            pltpu.VMEM((_SC_BATCH, n), jnp.float32),
            pltpu.VMEM((n, _SC_BATCH), jnp.float32),
            pltpu.VMEM((_SC_BATCH, n), jnp.float32),
            pltpu.SemaphoreType.DMA,
        ],
    )
    def sck(x_hbm, o_hbm, ibuf, tbuf, obuf, sem):
        wid = lax.axis_index("s") * nc + lax.axis_index("c")
        base = wid * rows_w

        lanes = jnp.arange(l, dtype=jnp.int32)
        one = jnp.full((l,), 1, jnp.int32)
        zero = jnp.full((l,), 0, jnp.int32)
        kvec = jnp.full((l,), k, jnp.int32)

        def count_below(mid_f):
            def chunk(j, acc):
                for u in range(unroll):
                    v = tbuf[j * unroll + u]
                    acc = acc + jnp.where(v < mid_f, one, zero)
                return acc
            return lax.fori_loop(0, n // unroll, chunk,
                                 jnp.zeros((l,), jnp.int32))

        def batch(b, carry):
            r0 = base + b * _SC_BATCH
            pltpu.sync_copy(x_hbm.at[pl.ds(r0, _SC_BATCH)], ibuf)

            def transpose(j, carry2):
                for u in range(unroll):
                    col = jnp.full((l,), j * unroll + u, jnp.int32)
                    tbuf[j * unroll + u] = plsc.load_gather(
                        ibuf, [lanes, col])
                return carry2
            lax.fori_loop(0, n // unroll, transpose, carry)

            c0 = count_below(jnp.zeros((l,), jnp.float32))
            p = jnp.where(c0 > kvec,
                          jnp.full((l,), _SIGN_BIT, jnp.int32), zero)

            def bitstep(i, p):
                mid = p + (one << (30 - i))
                mid_bits = jnp.where(mid < 0, mid ^ _REST_MASK, mid)
                mid_f = plsc.bitcast(mid_bits, jnp.float32)
                c = count_below(mid_f)
                return jnp.where(c > kvec, p, mid)

            p = lax.fori_loop(0, 31 - _LOW_BIT, bitstep, p)
            med_bits = jnp.where(p < 0, p ^ _REST_MASK, p)
            med = plsc.bitcast(med_bits, jnp.float32)

            def untranspose(j, carry2):
                for u in range(unroll):
                    col = jnp.full((l,), j * unroll + u, jnp.int32)
                    plsc.store_scatter(obuf, [lanes, col],
                                       tbuf[j * unroll + u] - med)
                return carry2
            lax.fori_loop(0, n // unroll, untranspose, carry)

            pltpu.sync_copy(obuf, o_hbm.at[pl.ds(r0, _SC_BATCH)])
            return carry

        lax.fori_loop(0, nb, batch, jnp.int32(0))

    return sck(x)


def kernel(x):
    out_sc = _sc_part(x)
    out_tc = _tc_part(x, _SC_ROWS)
    return jax.lax.dynamic_update_slice(out_tc, out_sc, (0, 0))
